# baseline (speedup 1.0000x reference)
import functools

import jax
import jax.numpy as jnp
from jax import lax
from jax.experimental import pallas as pl
from jax.experimental.pallas import tpu as pltpu
from jax.experimental.pallas import tpu_sc as plsc

B = 16384
F = 26
NROWS = B * F
EMB = 32
C_OUT = 128
N_TOK = 1000000

TBLK = 2048
SUB = 512
PACK = 4
N_PBLKS = -(-N_TOK // TBLK)
PACK_ROWS = N_PBLKS * SUB
LIN_ROWS = PACK_ROWS * PACK


def _pack_body(t_ref, o_ref):
    t = t_ref[...]
    cols = [t[:, j * SUB:(j + 1) * SUB].T for j in range(PACK)]
    o_ref[...] = jnp.concatenate(cols, axis=1)


def _pack_table(table_t):
    return pl.pallas_call(
        _pack_body,
        grid=(N_PBLKS,),
        in_specs=[pl.BlockSpec((EMB, TBLK), lambda i: (0, i))],
        out_specs=pl.BlockSpec((SUB, PACK * EMB), lambda i: (i, 0)),
        out_shape=jax.ShapeDtypeStruct((PACK_ROWS, PACK * EMB), jnp.float32),
    )(table_t)


NC = 2
NS = 16
NW = NC * NS
B_PER_W = NROWS // NW
CHUNK = 832
N_CHUNKS = B_PER_W // CHUNK
IDX_VECS = B_PER_W // 16


@functools.partial(
    pl.kernel,
    mesh=plsc.VectorSubcoreMesh(core_axis_name="c", subcore_axis_name="s"),
    out_type=jax.ShapeDtypeStruct((NROWS, EMB), jnp.float32),
    scratch_types=[
        pltpu.VMEM((B_PER_W,), jnp.int32),
        pltpu.VMEM((CHUNK, EMB), jnp.float32),
        pltpu.VMEM((CHUNK, EMB), jnp.float32),
        pltpu.SemaphoreType.DMA,
        pltpu.SemaphoreType.DMA,
    ],
    compiler_params=pltpu.CompilerParams(use_tc_tiling_on_sc=False),
)
def _sc_gather(idx_hbm, lin_hbm, out_hbm, idx_v, rows_v0, rows_v1, gsem, ssem):
    wid = lax.axis_index("s") * NC + lax.axis_index("c")
    base = wid * B_PER_W
    pltpu.sync_copy(idx_hbm.at[pl.ds(base, B_PER_W)], idx_v)

    def xform(k, _):
        t = idx_v[pl.ds(k * 16, 16)]
        g = (t & ~jnp.int32(2047)) | ((t & jnp.int32(511)) << 2) \
            | ((t >> 9) & jnp.int32(3))
        idx_v[pl.ds(k * 16, 16)] = g
        return _

    lax.fori_loop(0, IDX_VECS, xform, 0)

    row_bufs = (rows_v0, rows_v1)

    def start_gather(j):
        return pltpu.async_copy(
            lin_hbm.at[idx_v.at[pl.ds(j * CHUNK, CHUNK)]], row_bufs[j % 2], gsem
        )

    gathers = {0: start_gather(0)}
    scatters = {}
    for j in range(N_CHUNKS):
        if j + 1 < N_CHUNKS:
            if j - 1 >= 0:
                scatters.pop(j - 1).wait()
            gathers[j + 1] = start_gather(j + 1)
        gathers.pop(j).wait()
        scatters[j] = pltpu.async_copy(
            row_bufs[j % 2], out_hbm.at[pl.ds(base + j * CHUNK, CHUNK)], ssem
        )
    scatters.pop(N_CHUNKS - 1).wait()


RBLK = 1024
PR = NROWS // PACK
N_MBLKS = PR // RBLK


def _mm_body(e_ref, w_ref, b_ref, o_ref):
    o_ref[...] = (
        jnp.dot(e_ref[...], w_ref[...], preferred_element_type=jnp.float32)
        + b_ref[...]
    )


def _mm(emb_pack, w4, b4):
    return pl.pallas_call(
        _mm_body,
        grid=(N_MBLKS,),
        in_specs=[
            pl.BlockSpec((RBLK, PACK * EMB), lambda i: (i, 0)),
            pl.BlockSpec((PACK * EMB, PACK * C_OUT), lambda i: (0, 0)),
            pl.BlockSpec((1, PACK * C_OUT), lambda i: (0, 0)),
        ],
        out_specs=pl.BlockSpec((RBLK, PACK * C_OUT), lambda i: (i, 0)),
        out_shape=jax.ShapeDtypeStruct((PR, PACK * C_OUT), jnp.float32),
    )(emb_pack, w4, b4)


@jax.jit
def kernel(x, table, W_out, b_out):
    lin = _pack_table(table.T).reshape(LIN_ROWS, EMB)
    idx = x.T.reshape(-1).astype(jnp.int32)
    emb = _sc_gather(idx, lin)
    wt = W_out.T
    eye = jnp.eye(PACK, dtype=jnp.float32)
    w4 = (eye[:, None, :, None] * wt[None, :, None, :]).reshape(
        PACK * EMB, PACK * C_OUT
    )
    b4 = jnp.tile(b_out, PACK).reshape(1, PACK * C_OUT)
    out4 = _mm(emb.reshape(PR, PACK * EMB), w4, b4)
    return out4.reshape(F, B, C_OUT).transpose(1, 0, 2)
